# Initial kernel scaffold; baseline (speedup 1.0000x reference)
import jax
import jax.numpy as jnp
from jax.experimental import pallas as pl


def kernel(x, edge_index, batch, target, gat_W, att_src, att_dst, gat_bias, gcn_W, gcn_bias, fc_g1_W, fc_g1_b, fc_g2_W, fc_g2_b, emb_table, conv_W, conv_b, fc1_xt_W, fc1_xt_b, fc1_W, fc1_b, out_W, out_b):
    raise NotImplementedError("write your pallas kernel here")


# speedup vs baseline: 1.0278x; 1.0278x over previous
import functools

import jax
import jax.numpy as jnp
from jax.experimental import pallas as pl
from jax.experimental.pallas import tpu as pltpu

H = 10
C = 35


def _ceil_to(a, m):
    return (a + m - 1) // m * m


def _matmul_body(x_ref, w_ref, b_ref, o_ref, *, act):
    acc = jnp.dot(x_ref[...], w_ref[...], preferred_element_type=jnp.float32)
    acc = acc + b_ref[...]
    if act == "relu":
        acc = jnp.maximum(acc, 0.0)
    elif act == "sigmoid":
        acc = jax.nn.sigmoid(acc)
    o_ref[...] = acc


def matmul(x, w, b=None, act="none", bm=512):
    M, K = x.shape
    K2, N = w.shape
    assert K == K2
    Mp = _ceil_to(M, bm)
    Kp = _ceil_to(K, 128)
    Np = _ceil_to(N, 128)
    xp = jnp.zeros((Mp, Kp), jnp.float32).at[:M, :K].set(x)
    wp = jnp.zeros((Kp, Np), jnp.float32).at[:K, :N].set(w)
    bp = jnp.zeros((1, Np), jnp.float32)
    if b is not None:
        bp = bp.at[0, :N].set(b)
    out = pl.pallas_call(
        functools.partial(_matmul_body, act=act),
        grid=(Mp // bm,),
        in_specs=[
            pl.BlockSpec((bm, Kp), lambda i: (i, 0)),
            pl.BlockSpec((Kp, Np), lambda i: (0, 0)),
            pl.BlockSpec((1, Np), lambda i: (0, 0)),
        ],
        out_specs=pl.BlockSpec((bm, Np), lambda i: (i, 0)),
        out_shape=jax.ShapeDtypeStruct((Mp, Np), jnp.float32),
    )(xp, wp, bp)
    return out[:M, :N]


def kernel(x, edge_index, batch, target, gat_W, att_src, att_dst, gat_bias,
           gcn_W, gcn_bias, fc_g1_W, fc_g1_b, fc_g2_W, fc_g2_b,
           emb_table, conv_W, conv_b, fc1_xt_W, fc1_xt_b,
           fc1_W, fc1_b, out_W, out_b):
    N = x.shape[0]
    nb = target.shape[0]
    loops = jnp.arange(N, dtype=edge_index.dtype)
    ei = jnp.concatenate([edge_index, jnp.stack([loops, loops])], axis=1)
    src, dst = ei[0], ei[1]

    h = matmul(x, gat_W).reshape(N, H, C)
    a_src = jnp.sum(h * att_src, axis=-1)
    a_dst = jnp.sum(h * att_dst, axis=-1)
    alpha = a_src[src] + a_dst[dst]
    alpha = jax.nn.leaky_relu(alpha, negative_slope=0.2)
    ea = jnp.exp(alpha)
    denom = jax.ops.segment_sum(ea, dst, num_segments=N)
    att = ea / (denom[dst] + 1e-16)
    msg = h[src] * att[:, :, None]
    x1 = jax.ops.segment_sum(msg, dst, num_segments=N).reshape(N, H * C) + gat_bias
    x1 = jax.nn.relu(x1)

    deg = jax.ops.segment_sum(jnp.ones_like(src, dtype=x.dtype), dst, num_segments=N)
    dinv = jnp.where(deg > 0, jax.lax.rsqrt(deg), 0.0)
    norm = dinv[src] * dinv[dst]
    h2 = matmul(x1, gcn_W)
    x2 = jax.ops.segment_sum(h2[src] * norm[:, None], dst, num_segments=N) + gcn_bias
    x2 = jax.nn.relu(x2)

    gmax = jax.ops.segment_max(x2, batch, num_segments=nb)
    gmax = jnp.where(jnp.isfinite(gmax), gmax, 0.0)
    gsum = jax.ops.segment_sum(x2, batch, num_segments=nb)
    cnt = jax.ops.segment_sum(jnp.ones((N,), x.dtype), batch, num_segments=nb)
    gmean = gsum / jnp.clip(cnt, 1.0)[:, None]
    xg = jnp.concatenate([gmax, gmean], axis=1)
    xg = matmul(xg, fc_g1_W, fc_g1_b, act="relu", bm=256)
    xg = matmul(xg, fc_g2_W, fc_g2_b, bm=256)

    emb = emb_table[target]
    emb = jnp.transpose(emb, (0, 2, 1))
    conv = jax.lax.conv_general_dilated(
        emb, conv_W, window_strides=(1,), padding='VALID',
        dimension_numbers=('NCH', 'OIH', 'NCH')) + conv_b[None, :, None]
    xt = conv.reshape(nb, 16 * 35)
    xt = matmul(xt, fc1_xt_W, fc1_xt_b, bm=256)

    xc = jnp.concatenate([xg, xt], axis=1)
    xc = matmul(xc, fc1_W, fc1_b, act="relu", bm=256)
    out = matmul(xc, out_W, out_b, act="sigmoid", bm=256)
    return out, att


# speedup vs baseline: 14.4616x; 14.0710x over previous
import functools

import jax
import jax.numpy as jnp
from jax import lax
from jax.experimental import pallas as pl
from jax.experimental.pallas import tpu as pltpu
from jax.experimental.pallas import tpu_sc as plsc

H = 10
C = 35

NC = 2
NS = 16
LANE = 16

N_NODES = 50000
N_EDGES_TOT = 850000
NP2 = 50688
EP = 851968
FEAT = 384
KB = 512


def _ceil_to(a, m):
    return (a + m - 1) // m * m


def _matmul_body(x_ref, w_ref, b_ref, ib_ref, o_ref, *, act, act_in):
    xv = x_ref[...]
    if act_in:
        xv = jnp.maximum(xv + ib_ref[...], 0.0)
    acc = jnp.dot(xv, w_ref[...], preferred_element_type=jnp.float32)
    acc = acc + b_ref[...]
    if act == "relu":
        acc = jnp.maximum(acc, 0.0)
    elif act == "sigmoid":
        acc = jax.nn.sigmoid(acc)
    o_ref[...] = acc


def matmul(x, w, b=None, act="none", bm=512, in_bias=None, out_rows=None,
           out_cols=None):
    M, K = x.shape
    K2, N = w.shape
    Mp = _ceil_to(out_rows or M, bm)
    Kp = _ceil_to(K, 128)
    Np = _ceil_to(out_cols or N, 128)
    xp = x
    if x.shape != (Mp, Kp):
        xp = jnp.zeros((Mp, Kp), jnp.float32).at[:M, :K].set(x)
    wp = jnp.zeros((Kp, Np), jnp.float32).at[:K2, :N].set(w)
    bp = jnp.zeros((1, Np), jnp.float32)
    if b is not None:
        bp = bp.at[0, :N].set(b)
    ibp = jnp.zeros((1, Kp), jnp.float32)
    act_in = in_bias is not None
    if act_in:
        ibp = ibp.at[0, :in_bias.shape[0]].set(in_bias)
    out = pl.pallas_call(
        functools.partial(_matmul_body, act=act, act_in=act_in),
        grid=(Mp // bm,),
        in_specs=[
            pl.BlockSpec((bm, Kp), lambda i: (i, 0)),
            pl.BlockSpec((Kp, Np), lambda i: (0, 0)),
            pl.BlockSpec((1, Np), lambda i: (0, 0)),
            pl.BlockSpec((1, Kp), lambda i: (0, 0)),
        ],
        out_specs=pl.BlockSpec((bm, Np), lambda i: (i, 0)),
        out_shape=jax.ShapeDtypeStruct((Mp, Np), jnp.float32),
    )(xp, wp, bp, ibp)
    or_, oc_ = out_rows or M, out_cols or N
    return out[:or_, :oc_]


_MESH = dict(core_axis_name="c", subcore_axis_name="s",
             num_cores=NC, num_subcores=NS)
_ZR = NP2 // NS


def _k1_body(asrc_hbm, adst_hbm, src_hbm, dst_hbm, z_hbm,
             ea_hbm, denp_hbm,
             src_v, dst_v, ar_v, ad_v, acc, sem1, sem2):
    cid = lax.axis_index("c")
    sid = lax.axis_index("s")
    wid = cid * NS + sid
    q = EP // (NC * NS)
    pltpu.sync_copy(z_hbm, acc.at[pl.ds(sid * _ZR, _ZR)])
    plsc.subcore_barrier()

    def batch(b, carry):
        base = wid * q + b * KB
        pltpu.sync_copy(src_hbm.at[pl.ds(base, KB)], src_v)
        pltpu.sync_copy(dst_hbm.at[pl.ds(base, KB)], dst_v)
        cp1 = pltpu.async_copy(asrc_hbm.at[src_v], ar_v, sem1)
        cp2 = pltpu.async_copy(adst_hbm.at[dst_v], ad_v, sem2)
        cp1.wait()
        cp2.wait()

        def row(j, c2):
            a = ar_v[j, :] + ad_v[j, :]
            a = jnp.where(a >= 0.0, a, 0.2 * a)
            ar_v[j, :] = jnp.exp(a)
            return c2
        lax.fori_loop(0, KB, row, 0)
        pltpu.sync_copy(ar_v, ea_hbm.at[pl.ds(base, KB)])
        pltpu.sync_copy(ar_v, acc.at[dst_v], add=True)
        return carry
    lax.fori_loop(0, q // KB, batch, 0)
    plsc.subcore_barrier()
    pltpu.sync_copy(acc.at[pl.ds(sid * _ZR, _ZR)],
                    denp_hbm.at[cid, pl.ds(sid * _ZR, _ZR)])


def edge_softmax_denom(asrc_p, adst_p, src_e, dst_e, zrow):
    return pl.kernel(
        _k1_body,
        out_type=[jax.ShapeDtypeStruct((EP, LANE), jnp.float32),
                  jax.ShapeDtypeStruct((NC, NP2, LANE), jnp.float32)],
        mesh=plsc.VectorSubcoreMesh(**_MESH),
        scratch_types=[
            pltpu.VMEM((KB,), jnp.int32),
            pltpu.VMEM((KB,), jnp.int32),
            pltpu.VMEM((KB, LANE), jnp.float32),
            pltpu.VMEM((KB, LANE), jnp.float32),
            pltpu.VMEM_SHARED((NP2, LANE), jnp.float32),
            pltpu.SemaphoreType.DMA,
            pltpu.SemaphoreType.DMA,
        ],
        compiler_params=pltpu.CompilerParams(use_tc_tiling_on_sc=False),
        name="sc_gat_edge_denom",
    )(asrc_p, adst_p, src_e, dst_e, zrow)


def _k2_body(ea_hbm, aux_hbm, src_hbm, dst_hbm,
             att_hbm,
             src_v, dst_v, ea_v, nd_v, ns_v, sem1, sem2, sem3):
    cid = lax.axis_index("c")
    sid = lax.axis_index("s")
    wid = cid * NS + sid
    q = EP // (NC * NS)
    lane_is10 = lax.iota(jnp.int32, LANE) == 10

    def batch(b, carry):
        base = wid * q + b * KB
        pltpu.sync_copy(src_hbm.at[pl.ds(base, KB)], src_v)
        pltpu.sync_copy(dst_hbm.at[pl.ds(base, KB)], dst_v)
        cp1 = pltpu.async_copy(ea_hbm.at[pl.ds(base, KB)], ea_v, sem1)
        cp2 = pltpu.async_copy(aux_hbm.at[dst_v], nd_v, sem2)
        cp3 = pltpu.async_copy(aux_hbm.at[src_v], ns_v, sem3)
        cp1.wait()
        cp2.wait()
        cp3.wait()

        def row(j, c2):
            sel = jnp.where(lane_is10, ns_v[j, :], 1.0)
            ea_v[j, :] = ea_v[j, :] * nd_v[j, :] * sel
            return c2
        lax.fori_loop(0, KB, row, 0)
        pltpu.sync_copy(ea_v, att_hbm.at[pl.ds(base, KB)])
        return carry
    lax.fori_loop(0, q // KB, batch, 0)


def edge_att_norm(ea_e, aux_p, src_e, dst_e):
    return pl.kernel(
        _k2_body,
        out_type=jax.ShapeDtypeStruct((EP, LANE), jnp.float32),
        mesh=plsc.VectorSubcoreMesh(**_MESH),
        scratch_types=[
            pltpu.VMEM((KB,), jnp.int32),
            pltpu.VMEM((KB,), jnp.int32),
            pltpu.VMEM((KB, LANE), jnp.float32),
            pltpu.VMEM((KB, LANE), jnp.float32),
            pltpu.VMEM((KB, LANE), jnp.float32),
            pltpu.SemaphoreType.DMA,
            pltpu.SemaphoreType.DMA,
            pltpu.SemaphoreType.DMA,
        ],
        compiler_params=pltpu.CompilerParams(use_tc_tiling_on_sc=False),
        name="sc_gat_edge_att",
    )(ea_e, aux_p, src_e, dst_e)


RNG = 2112
NRANGE = 24
TRASH = RNG
GB = 128
CAP = 4096
NCHUNK = FEAT // LANE
_ZR2 = (RNG + LANE) // NS
_WR = RNG // NS


def _agg_body(src_hbm, dst_hbm, att_hbm, h_hbm, hidx_hbm, z_hbm,
              out_hbm,
              src_v, dst_v, sel_eid, sel_src, sel_dstl, idx2d,
              attb, hb, hidx_v, acc, sem1, sem2, *, head_mode):
    cid = lax.axis_index("c")
    sid = lax.axis_index("s")
    q16 = EP // NS
    lanes = lax.iota(jnp.int32, LANE)
    pltpu.sync_copy(hidx_hbm, hidx_v)
    hidx_chunks = [hidx_v[pl.ds(k * LANE, LANE)] for k in range(NCHUNK)]
    trash_splat = jnp.full((LANE,), TRASH, jnp.int32)
    zero_splat = jnp.zeros((LANE,), jnp.int32)
    lane10 = jnp.full((LANE,), 10, jnp.int32)

    def do_range(r, carry):
        @pl.when(lax.rem(r, 2) == cid)
        def _():
            lo = r * RNG
            pltpu.sync_copy(z_hbm, acc.at[pl.ds(sid * _ZR2, _ZR2)])
            plsc.subcore_barrier()

            def scan_batch(b, cur):
                base = sid * q16 + b * KB
                pltpu.sync_copy(src_hbm.at[pl.ds(base, KB)], src_v)
                pltpu.sync_copy(dst_hbm.at[pl.ds(base, KB)], dst_v)

                def chunk(k, cur2):
                    dv = dst_v[pl.ds(k * LANE, LANE)]
                    sv = src_v[pl.ds(k * LANE, LANE)]
                    m = (dv >= lo) & (dv < lo + RNG)
                    eidv = lanes + (base + k * LANE)
                    cc = jnp.minimum(cur2, CAP - LANE)
                    plsc.store_compressed(sel_eid.at[pl.ds(cc, LANE)], eidv, mask=m)
                    plsc.store_compressed(sel_src.at[pl.ds(cc, LANE)], sv, mask=m)
                    plsc.store_compressed(sel_dstl.at[pl.ds(cc, LANE)], dv - lo, mask=m)
                    npop = plsc.all_reduce_population_count(m)
                    return cc + jnp.max(npop)
                return lax.fori_loop(0, KB // LANE, chunk, cur)
            cursor = lax.fori_loop(0, EP // NS // KB, scan_batch, jnp.int32(0))
            ng = (cursor + GB - 1) // GB

            def fill(k, c3):
                idx = k * LANE + lanes
                m = idx >= cursor
                plsc.store_scatter(sel_dstl, [idx], trash_splat, mask=m)
                plsc.store_scatter(sel_src, [idx], zero_splat, mask=m)
                plsc.store_scatter(sel_eid, [idx], zero_splat, mask=m)
                return c3
            lax.fori_loop(cursor // LANE, ng * (GB // LANE), fill, 0)

            def g2(g, c4):
                def cp(k, c5):
                    idx2d[g, pl.ds(k * LANE, LANE)] = \
                        sel_dstl[pl.ds(g * GB + k * LANE, LANE)]
                    return c5
                return lax.fori_loop(0, GB // LANE, cp, c4)
            lax.fori_loop(0, ng, g2, 0)

            def flush(g, c6):
                cp1 = pltpu.async_copy(
                    att_hbm.at[sel_eid.at[pl.ds(g * GB, GB)]], attb, sem1)
                cp2 = pltpu.async_copy(
                    h_hbm.at[sel_src.at[pl.ds(g * GB, GB)]], hb, sem2)
                cp1.wait()
                cp2.wait()

                def row(j, c7):
                    jj = jnp.full((LANE,), j, jnp.int32)
                    if head_mode:
                        for k in range(NCHUNK):
                            sc = plsc.load_gather(attb, [jj, hidx_chunks[k]])
                            hb[j, pl.ds(k * LANE, LANE)] = \
                                hb[j, pl.ds(k * LANE, LANE)] * sc
                    else:
                        sc = plsc.load_gather(attb, [jj, lane10])
                        for k in range(NCHUNK):
                            hb[j, pl.ds(k * LANE, LANE)] = \
                                hb[j, pl.ds(k * LANE, LANE)] * sc
                    return c7
                lax.fori_loop(0, GB, row, 0)
                pltpu.sync_copy(hb, acc.at[idx2d.at[g]], add=True)
                return c6
            lax.fori_loop(0, ng, flush, 0)
            plsc.subcore_barrier()
            pltpu.sync_copy(acc.at[pl.ds(sid * _WR, _WR)],
                            out_hbm.at[pl.ds(lo + sid * _WR, _WR)])
            plsc.subcore_barrier()
        return carry
    lax.fori_loop(0, NRANGE, do_range, 0)


def edge_aggregate(src_e, dst_e, att_e, h_p, hidx, zrows, head_mode):
    return pl.kernel(
        functools.partial(_agg_body, head_mode=head_mode),
        out_type=jax.ShapeDtypeStruct((NP2, FEAT), jnp.float32),
        mesh=plsc.VectorSubcoreMesh(**_MESH),
        scratch_types=[
            pltpu.VMEM((KB,), jnp.int32),
            pltpu.VMEM((KB,), jnp.int32),
            pltpu.VMEM((CAP,), jnp.int32),
            pltpu.VMEM((CAP,), jnp.int32),
            pltpu.VMEM((CAP,), jnp.int32),
            pltpu.VMEM((CAP // GB, GB), jnp.int32),
            pltpu.VMEM((GB, LANE), jnp.float32),
            pltpu.VMEM((GB, FEAT), jnp.float32),
            pltpu.VMEM((FEAT,), jnp.int32),
            pltpu.VMEM_SHARED((RNG + LANE, FEAT), jnp.float32),
            pltpu.SemaphoreType.DMA,
            pltpu.SemaphoreType.DMA,
        ],
        compiler_params=pltpu.CompilerParams(use_tc_tiling_on_sc=False,
                                             needs_layout_passes=False),
        name="sc_edge_aggregate" + ("_gat" if head_mode else "_gcn"),
    )(src_e, dst_e, att_e, h_p, hidx, zrows)


def kernel(x, edge_index, batch, target, gat_W, att_src, att_dst, gat_bias,
           gcn_W, gcn_bias, fc_g1_W, fc_g1_b, fc_g2_W, fc_g2_b,
           emb_table, conv_W, conv_b, fc1_xt_W, fc1_xt_b,
           fc1_W, fc1_b, out_W, out_b):
    N = x.shape[0]
    nb = target.shape[0]
    loops = jnp.arange(N, dtype=edge_index.dtype)
    ei = jnp.concatenate([edge_index, jnp.stack([loops, loops])], axis=1)
    src = ei[0].astype(jnp.int32)
    dst = ei[1].astype(jnp.int32)
    src_e = jnp.full((EP,), 0, jnp.int32).at[:N_EDGES_TOT].set(src)
    dst_e = jnp.full((EP,), N, jnp.int32).at[:N_EDGES_TOT].set(dst)

    h = matmul(x, gat_W, out_rows=NP2, out_cols=FEAT)
    h3 = h[:, :H * C].reshape(NP2, H, C)
    asrc_p = jnp.zeros((NP2, LANE), jnp.float32).at[:, :H].set(
        jnp.sum(h3 * att_src, axis=-1))
    adst_p = jnp.zeros((NP2, LANE), jnp.float32).at[:, :H].set(
        jnp.sum(h3 * att_dst, axis=-1))

    zrow = jnp.zeros((_ZR, LANE), jnp.float32)
    ea_e, denp = edge_softmax_denom(asrc_p, adst_p, src_e, dst_e, zrow)
    denom = denp[0] + denp[1]
    rden = 1.0 / (denom + 1e-16)
    deg = denom[:, 10]
    dinv = jnp.where(deg > 0, lax.rsqrt(deg), 0.0)
    aux = rden.at[:, 10].set(dinv)
    aux = aux.at[:, 11:].set(0.0)

    att_e = edge_att_norm(ea_e, aux, src_e, dst_e)
    att = att_e[:N_EDGES_TOT, :H]

    hidx = jnp.concatenate([
        jnp.repeat(jnp.arange(H, dtype=jnp.int32), C),
        jnp.full((FEAT - H * C,), 15, jnp.int32)])
    zrows = jnp.zeros((_ZR2, FEAT), jnp.float32)
    x1p = edge_aggregate(src_e, dst_e, att_e, h, hidx, zrows, True)

    h2p = matmul(x1p, gcn_W, in_bias=gat_bias,
                 out_rows=NP2, out_cols=FEAT)
    x2p = edge_aggregate(src_e, dst_e, att_e, h2p, hidx, zrows, False)
    x2 = jax.nn.relu(x2p[:N, :H * C] + gcn_bias)

    gmax = jax.ops.segment_max(x2, batch, num_segments=nb)
    gmax = jnp.where(jnp.isfinite(gmax), gmax, 0.0)
    gsum = jax.ops.segment_sum(x2, batch, num_segments=nb)
    cnt = jax.ops.segment_sum(jnp.ones((N,), x.dtype), batch, num_segments=nb)
    gmean = gsum / jnp.clip(cnt, 1.0)[:, None]
    xg = jnp.concatenate([gmax, gmean], axis=1)
    xg = matmul(xg, fc_g1_W, fc_g1_b, act="relu", bm=256)
    xg = matmul(xg, fc_g2_W, fc_g2_b, bm=256)

    emb = emb_table[target]
    emb = jnp.transpose(emb, (0, 2, 1))
    conv = jax.lax.conv_general_dilated(
        emb, conv_W, window_strides=(1,), padding='VALID',
        dimension_numbers=('NCH', 'OIH', 'NCH')) + conv_b[None, :, None]
    xt = conv.reshape(nb, 16 * 35)
    xt = matmul(xt, fc1_xt_W, fc1_xt_b, bm=256)

    xc = jnp.concatenate([xg, xt], axis=1)
    xc = matmul(xc, fc1_W, fc1_b, act="relu", bm=256)
    out = matmul(xc, out_W, out_b, act="sigmoid", bm=256)
    return out, att
